# baseline (speedup 1.0000x reference)
import dataclasses
import functools

import jax
import jax.numpy as jnp
from jax import lax
from jax.experimental import pallas as pl
from jax.experimental.pallas import tpu as pltpu
from jax.experimental.pallas import tpu_sc as plsc

BATCH = 16384
EMBED = 32

NC = 2
NS = 16
NW = NC * NS
BPW = BATCH // NW
CHUNK = 128
NCHUNK = BPW // CHUNK


def _sc_gather(cust, prod, ip, ic):
    mesh = plsc.VectorSubcoreMesh(core_axis_name="c", subcore_axis_name="s")
    cp = dataclasses.replace(pltpu.CompilerParams(), use_tc_tiling_on_sc=False)

    @functools.partial(
        pl.kernel,
        mesh=mesh,
        compiler_params=cp,
        out_type=[
            jax.ShapeDtypeStruct((BATCH, EMBED), jnp.float32),
            jax.ShapeDtypeStruct((BATCH, EMBED), jnp.float32),
        ],
        scratch_types=[
            pltpu.VMEM((NCHUNK, CHUNK), jnp.int32),
            pltpu.VMEM((NCHUNK, CHUNK), jnp.int32),
            pltpu.VMEM((BPW, EMBED), jnp.float32),
            pltpu.VMEM((BPW, EMBED), jnp.float32),
            pltpu.SemaphoreType.DMA,
            pltpu.SemaphoreType.DMA,
        ],
    )
    def k(cust_hbm, prod_hbm, ip_hbm, ic_hbm, op_hbm, oc_hbm,
          ipv, icv, pv, cv, semp, semc):
        wid = lax.axis_index("s") * NC + lax.axis_index("c")
        pltpu.sync_copy(ip_hbm.at[wid], ipv)
        pltpu.sync_copy(ic_hbm.at[wid], icv)
        copies = []
        for j in range(NCHUNK):
            dst = pl.ds(j * CHUNK, CHUNK)
            copies.append(
                pltpu.async_copy(cust_hbm.at[ipv.at[j]], pv.at[dst], semp))
            copies.append(
                pltpu.async_copy(prod_hbm.at[icv.at[j]], cv.at[dst], semc))
        for c in copies:
            c.wait()
        base = wid * BPW
        pltpu.sync_copy(pv, op_hbm.at[pl.ds(base, BPW)])
        pltpu.sync_copy(cv, oc_hbm.at[pl.ds(base, BPW)])

    return k(cust, prod, ip, ic)


_MLP_BS = 2048


def _mlp_body(p_ref, c_ref, wp, bp, wc, bc, w2a, w2b, b2, wo, bo, o_ref):
    f32 = jnp.float32
    hp = jnp.maximum(
        jnp.dot(p_ref[...], wp[...], preferred_element_type=f32) + bp[...], 0.0)
    hc = jnp.maximum(
        jnp.dot(c_ref[...], wc[...], preferred_element_type=f32) + bc[...], 0.0)
    h2 = jnp.maximum(
        jnp.dot(hp, w2a[...], preferred_element_type=f32)
        + jnp.dot(hc, w2b[...], preferred_element_type=f32) + b2[...], 0.0)
    z = jnp.dot(h2, wo[...], preferred_element_type=f32) + bo[...]
    o_ref[...] = jax.nn.sigmoid(z)


def _mlp(pvec, cvec, Wp, bp, Wc, bc, W2a, W2b, b2, Wo, bo):
    grid = (BATCH // _MLP_BS,)
    full = lambda a: pl.BlockSpec(a.shape, lambda i: (0, 0))
    return pl.pallas_call(
        _mlp_body,
        grid=grid,
        in_specs=[
            pl.BlockSpec((_MLP_BS, EMBED), lambda i: (i, 0)),
            pl.BlockSpec((_MLP_BS, EMBED), lambda i: (i, 0)),
            full(Wp), full(bp), full(Wc), full(bc),
            full(W2a), full(W2b), full(b2), full(Wo), full(bo),
        ],
        out_specs=pl.BlockSpec((_MLP_BS, 1), lambda i: (i, 0)),
        out_shape=jax.ShapeDtypeStruct((BATCH, 1), jnp.float32),
    )(pvec, cvec, Wp, bp, Wc, bc, W2a, W2b, b2, Wo, bo)


def kernel(X, encoded_customers, encoded_products, W_prod, b_prod,
           W_cust, b_cust, W_fc2, b_fc2, W_out, b_out):
    ip = X[:, 0].astype(jnp.int32).reshape(NW, NCHUNK, CHUNK)
    ic = X[:, 1].astype(jnp.int32).reshape(NW, NCHUNK, CHUNK)
    product_vec, customer_vec = _sc_gather(encoded_customers,
                                           encoded_products, ip, ic)
    out = _mlp(
        product_vec, customer_vec,
        W_prod, b_prod.reshape(1, 10),
        W_cust, b_cust.reshape(1, 10),
        W_fc2[:10], W_fc2[10:], b_fc2.reshape(1, 10),
        W_out, b_out.reshape(1, 1),
    )
    return out
